# baseline (speedup 1.0000x reference)
import jax
import jax.numpy as jnp
from jax import lax
from jax.experimental import pallas as pl
from jax.experimental.pallas import tpu as pltpu

NUM_EXPERTS = 16
TOP_K = 2
_NEG_INF = float("-inf")
TB = 1024


def _router_block(x_ref, w_ref, probs_ref, tkl_ref, tki_ref, mask_ref):
    xb = x_ref[...]
    w = w_ref[...]
    logits = lax.dot_general(w, xb, (((1,), (1,)), ((), ())),
                             preferred_element_type=jnp.float32)
    iota = lax.broadcasted_iota(jnp.int32, logits.shape, 0)
    m1 = jnp.max(logits, axis=0, keepdims=True)
    i1 = jnp.min(jnp.where(logits == m1, iota, NUM_EXPERTS),
                 axis=0, keepdims=True)
    sel1 = iota == i1
    masked = jnp.where(sel1, _NEG_INF, logits)
    m2 = jnp.max(masked, axis=0, keepdims=True)
    i2 = jnp.min(jnp.where(masked == m2, iota, NUM_EXPERTS),
                 axis=0, keepdims=True)
    sel2 = iota == i2
    keep = sel1 | sel2
    e = jnp.exp(logits - m1)
    denom = 1.0 + jnp.exp(m2 - m1)
    probs_ref[...] = jnp.where(keep, e / denom, 0.0).T
    tkl_ref[...] = jnp.concatenate([m1, m2], axis=0).T
    tki_ref[...] = jnp.concatenate([i1, i2], axis=0).T
    mask_ref[0] = sel1.astype(jnp.float32).T
    mask_ref[1] = sel2.astype(jnp.float32).T


def kernel(x, W_gate):
    Bsz, Tlen, D = x.shape
    E = W_gate.shape[0]
    nt = Bsz * Tlen
    xf = x.reshape(nt, D)
    grid = (nt // TB,)
    probs, tkl, tki, mask = pl.pallas_call(
        _router_block,
        grid=grid,
        in_specs=[
            pl.BlockSpec((TB, D), lambda i: (i, 0)),
            pl.BlockSpec((E, D), lambda i: (0, 0)),
        ],
        out_specs=[
            pl.BlockSpec((TB, E), lambda i: (i, 0)),
            pl.BlockSpec((TB, TOP_K), lambda i: (i, 0)),
            pl.BlockSpec((TB, TOP_K), lambda i: (i, 0)),
            pl.BlockSpec((TOP_K, TB, E), lambda i: (0, i, 0)),
        ],
        out_shape=[
            jax.ShapeDtypeStruct((nt, E), jnp.float32),
            jax.ShapeDtypeStruct((nt, TOP_K), jnp.float32),
            jax.ShapeDtypeStruct((nt, TOP_K), jnp.int32),
            jax.ShapeDtypeStruct((TOP_K, nt, E), jnp.float32),
        ],
        compiler_params=pltpu.CompilerParams(
            dimension_semantics=("parallel",)),
    )(xf, W_gate)
    return (probs.reshape(Bsz, Tlen, E),
            tkl.reshape(Bsz, Tlen, TOP_K),
            tki.reshape(Bsz, Tlen, TOP_K),
            mask)


# speedup vs baseline: 1.7291x; 1.7291x over previous
import jax
import jax.numpy as jnp
from jax import lax
from jax.experimental import pallas as pl
from jax.experimental.pallas import tpu as pltpu

NUM_EXPERTS = 16
TOP_K = 2
_NEG_INF = float("-inf")
TB = 1024


def _router_block(x_ref, w_ref, probs_ref, tk_ref, mask_ref):
    xb = x_ref[...]
    w = w_ref[...]
    logits = lax.dot_general(w, xb, (((1,), (1,)), ((), ())),
                             preferred_element_type=jnp.float32)
    iota = lax.broadcasted_iota(jnp.int32, logits.shape, 0)
    m1 = jnp.max(logits, axis=0, keepdims=True)
    i1 = jnp.min(jnp.where(logits == m1, iota, NUM_EXPERTS),
                 axis=0, keepdims=True)
    sel1 = iota == i1
    masked = jnp.where(sel1, _NEG_INF, logits)
    m2 = jnp.max(masked, axis=0, keepdims=True)
    i2 = jnp.min(jnp.where(masked == m2, iota, NUM_EXPERTS),
                 axis=0, keepdims=True)
    sel2 = iota == i2
    keep = sel1 | sel2
    e = jnp.exp(logits - m1)
    denom = 1.0 + jnp.exp(m2 - m1)
    probs_ref[...] = jnp.where(keep, e / denom, 0.0)
    tk_ref[...] = jnp.concatenate(
        [m1, m2,
         lax.bitcast_convert_type(i1, jnp.float32),
         lax.bitcast_convert_type(i2, jnp.float32),
         jnp.zeros((4, logits.shape[1]), jnp.float32)], axis=0)
    mask_ref[0] = sel1.astype(jnp.float32)
    mask_ref[1] = sel2.astype(jnp.float32)


def kernel(x, W_gate):
    Bsz, Tlen, D = x.shape
    E = W_gate.shape[0]
    nt = Bsz * Tlen
    xf = x.reshape(nt, D)
    grid = (nt // TB,)
    probsT, tk, maskT = pl.pallas_call(
        _router_block,
        grid=grid,
        in_specs=[
            pl.BlockSpec((TB, D), lambda i: (i, 0)),
            pl.BlockSpec((E, D), lambda i: (0, 0)),
        ],
        out_specs=[
            pl.BlockSpec((E, TB), lambda i: (0, i)),
            pl.BlockSpec((8, TB), lambda i: (0, i)),
            pl.BlockSpec((TOP_K, E, TB), lambda i: (0, 0, i)),
        ],
        out_shape=[
            jax.ShapeDtypeStruct((E, nt), jnp.float32),
            jax.ShapeDtypeStruct((8, nt), jnp.float32),
            jax.ShapeDtypeStruct((TOP_K, E, nt), jnp.float32),
        ],
        compiler_params=pltpu.CompilerParams(
            dimension_semantics=("parallel",)),
    )(xf, W_gate)
    probs = probsT.T.reshape(Bsz, Tlen, E)
    tkl = tk[0:TOP_K].T.reshape(Bsz, Tlen, TOP_K)
    tki = lax.bitcast_convert_type(
        tk[TOP_K:2 * TOP_K], jnp.int32).T.reshape(Bsz, Tlen, TOP_K)
    mask = maskT.transpose(0, 2, 1)
    return probs, tkl, tki, mask
